# baseline (speedup 1.0000x reference)
import jax
import jax.numpy as jnp
from jax import lax
from jax.experimental import pallas as pl
from jax.experimental.pallas import tpu as pltpu
from jax.experimental.pallas import tpu_sc as plsc

B = 128
L = 1025
D = 192
SEQ = 1024
NR = 256
NM = SEQ - NR
NB = 1024
LN = 16
CH = SEQ // LN
NC, NS = 2, 16
NW = NC * NS
RPW = B // NW


def _sc_sort_body(noise_hbm, out_remain, out_masked, out_revert,
                  ka, va, kb, vb, hist, offs, revert_v):
    wid = lax.axis_index("s") * NC + lax.axis_index("c")

    def row_body(r, carry):
        b = wid * RPW + r
        pltpu.sync_copy(noise_hbm.at[pl.ds(b * SEQ, SEQ)], ka)

        def init_chunk(c, carry):
            va[pl.ds(c * LN, LN)] = lax.iota(jnp.int32, LN) + c * LN
            return carry
        lax.fori_loop(0, CH, init_chunk, 0)

        for p, (ks, vs, kd, vd) in enumerate(
            ((ka, va, kb, vb), (kb, vb, ka, va), (ka, va, kb, vb))):
            shift = 10 * p

            def clear_chunk(c, carry):
                hist[pl.ds(c * LN, LN)] = jnp.zeros((LN,), jnp.int32)
                return carry
            lax.fori_loop(0, NB // LN, clear_chunk, 0)

            def hist_chunk(c, carry, ks=ks, shift=shift):
                k16 = ks[pl.ds(c * LN, LN)]
                d = (k16 >> shift) & (NB - 1)
                cnt, last = plsc.scan_count(d)
                plsc.addupdate_scatter(hist, [d], cnt, mask=last)
                return carry
            lax.fori_loop(0, CH, hist_chunk, 0)

            def scan_chunk(c, run):
                h = hist[pl.ds(c * LN, LN)]
                cs = plsc.cumsum(h)
                offs[pl.ds(c * LN, LN)] = cs - h + run
                return run + jnp.sum(h)
            lax.fori_loop(0, NB // LN, scan_chunk, jnp.int32(0))

            def perm_chunk(c, carry, ks=ks, vs=vs, kd=kd, vd=vd, shift=shift):
                k16 = ks[pl.ds(c * LN, LN)]
                v16 = vs[pl.ds(c * LN, LN)]
                d = (k16 >> shift) & (NB - 1)
                cnt, last = plsc.scan_count(d)
                base = plsc.load_gather(offs, [d])
                pos = base + cnt - 1
                plsc.store_scatter(kd, [pos], k16)
                plsc.store_scatter(vd, [pos], v16)
                plsc.addupdate_scatter(offs, [d], cnt, mask=last)
                return carry
            lax.fori_loop(0, CH, perm_chunk, 0)

        def rev_chunk(c, carry):
            v16 = vb[pl.ds(c * LN, LN)]
            plsc.store_scatter(revert_v, [v16], lax.iota(jnp.int32, LN) + c * LN)
            return carry
        lax.fori_loop(0, CH, rev_chunk, 0)

        pltpu.sync_copy(revert_v, out_revert.at[pl.ds(b * SEQ, SEQ)])
        pltpu.sync_copy(vb.at[pl.ds(0, NR)], out_remain.at[pl.ds(b * NR, NR)])
        pltpu.sync_copy(vb.at[pl.ds(NR, NM)], out_masked.at[pl.ds(b * NM, NM)])
        return carry

    lax.fori_loop(0, RPW, row_body, 0)


DP = 256


def _tc_transpose_body(in_ref, out_ref):
    x = in_ref[...]
    y = jnp.transpose(x, (0, 2, 1))
    pad = jnp.zeros(y.shape[:2] + (DP - D,), jnp.float32)
    out_ref[...] = jnp.concatenate([y, pad], axis=2)


KP = 264


def _sc_gather_body(data_hbm, remain_hbm, out_data, idx_v,
                    mi_a, mi_b, mi2_a, mi2_b, rv_a, rv_b,
                    semg_a, semg_b, semw_a, semw_b):
    wid = lax.axis_index("s") * NC + lax.axis_index("c")
    b0 = wid * RPW
    pltpu.sync_copy(remain_hbm.at[pl.ds(b0 * NR, RPW * NR)], idx_v)

    mi_ = [mi_a, mi_b]
    mi2_ = [mi2_a, mi2_b]
    iota = lax.iota(jnp.int32, LN)
    zeros = jnp.zeros((LN,), jnp.int32)

    def build_and_issue(r):
        buf = r % 2
        b = b0 + r
        bvec = zeros + b
        mi2_[buf][...] = bvec
        plsc.store_scatter(mi_[buf], [zeros, zeros], bvec, mask=iota == 0)

        def mi_chunk(c, carry, buf=buf, b=b, r=r):
            idx16 = idx_v[pl.ds(r * NR + c * LN, LN)]
            g16 = ((idx16 + 1) << 7) + b
            pos = iota + (c * LN + 1)
            r16 = pos >> 7
            c16 = pos & 127
            plsc.store_scatter(mi_[buf], [r16, c16], g16, mask=pos <= 255)
            plsc.store_scatter(mi2_[buf], [jnp.maximum(pos - NR, 0)], g16,
                              mask=pos == NR)
            return carry
        lax.fori_loop(0, NR // LN, mi_chunk, 0)
        cp0 = pltpu.async_copy(data_hbm.at[mi_[buf].at[0]],
                               rv_a, semg_a)
        cp1 = pltpu.async_copy(data_hbm.at[mi_[buf].at[1]],
                               rv_b.at[pl.ds(0, 128)], semg_b)
        cp2 = pltpu.async_copy(data_hbm.at[mi2_[buf].at[pl.ds(0, 8)]],
                               rv_b.at[pl.ds(128, 8)], semg_b)
        return cp0, (cp1, cp2)

    wa = wb = None
    for r in range(RPW):
        b = b0 + r
        buf = r % 2
        if wa is not None:
            wa.wait()
        if wb is not None:
            wb.wait()
        ga, gb = build_and_issue(r)
        ga.wait()
        wa = pltpu.async_copy(rv_a, out_data.at[b].at[pl.ds(0, 128)], semw_a)
        for d in gb:
            d.wait()
        wb = pltpu.async_copy(rv_b, out_data.at[b].at[pl.ds(128, KP - 128)],
                              semw_b)
    wa.wait()
    wb.wait()


def kernel(data, noise):
    noise_f = lax.bitcast_convert_type(noise, jnp.int32).reshape(B * SEQ)

    mesh = plsc.VectorSubcoreMesh(core_axis_name="c", subcore_axis_name="s")
    out_type = [
        jax.ShapeDtypeStruct((B * NR,), jnp.int32),
        jax.ShapeDtypeStruct((B * NM,), jnp.int32),
        jax.ShapeDtypeStruct((B * SEQ,), jnp.int32),
    ]
    scratch = [
        pltpu.VMEM((SEQ,), jnp.int32),
        pltpu.VMEM((SEQ,), jnp.int32),
        pltpu.VMEM((SEQ,), jnp.int32),
        pltpu.VMEM((SEQ,), jnp.int32),
        pltpu.VMEM((NB,), jnp.int32),
        pltpu.VMEM((NB,), jnp.int32),
        pltpu.VMEM((SEQ,), jnp.int32),
    ]
    sort_run = pl.kernel(_sc_sort_body, out_type=out_type, mesh=mesh,
                         scratch_types=scratch,
                         compiler_params=pltpu.CompilerParams(
                             needs_layout_passes=False,
                             use_tc_tiling_on_sc=False))
    orem, omask, orev = sort_run(noise_f)

    remain_idx = orem.reshape(B, NR)
    masked_idx = omask.reshape(B, NM)
    revert_idx = orev.reshape(B, SEQ)

    data_t = jnp.transpose(data, (1, 2, 0))
    J = 41
    data_rows = pl.pallas_call(
        _tc_transpose_body,
        grid=(L // J,),
        in_specs=[pl.BlockSpec((J, D, B), lambda j: (j, 0, 0))],
        out_specs=pl.BlockSpec((J, B, DP), lambda j: (j, 0, 0)),
        out_shape=jax.ShapeDtypeStruct((L, B, DP), jnp.float32),
    )(data_t).reshape(L * B, DP)

    gather_run = pl.kernel(
        _sc_gather_body,
        out_type=jax.ShapeDtypeStruct((B, KP, DP), jnp.float32),
        mesh=mesh,
        scratch_types=[
            pltpu.VMEM((RPW * NR,), jnp.int32),
            pltpu.VMEM((2, 128), jnp.int32),
            pltpu.VMEM((2, 128), jnp.int32),
            pltpu.VMEM((LN,), jnp.int32),
            pltpu.VMEM((LN,), jnp.int32),
            pltpu.VMEM((128, DP), jnp.float32),
            pltpu.VMEM((KP - 128, DP), jnp.float32),
            pltpu.SemaphoreType.DMA,
            pltpu.SemaphoreType.DMA,
            pltpu.SemaphoreType.DMA,
            pltpu.SemaphoreType.DMA,
        ],
        compiler_params=pltpu.CompilerParams(
            needs_layout_passes=False,
            use_tc_tiling_on_sc=True))
    out_pad = gather_run(data_rows, orem)

    total_remain_data = out_pad[:, :NR + 1, :D]

    total_remain_padding_mask = jnp.ones((B, NR + 1), jnp.float32)
    revert_padding_mask = jnp.ones((B, L), jnp.float32)
    return (total_remain_data, remain_idx, masked_idx, revert_idx,
            total_remain_padding_mask, revert_padding_mask)


# speedup vs baseline: 1.0054x; 1.0054x over previous
import jax
import jax.numpy as jnp
from jax import lax
from jax.experimental import pallas as pl
from jax.experimental.pallas import tpu as pltpu
from jax.experimental.pallas import tpu_sc as plsc

B = 128
L = 1025
D = 192
SEQ = 1024
NR = 256
NM = SEQ - NR
NB = 1024
LN = 16
CH = SEQ // LN
NC, NS = 2, 16
NW = NC * NS
RPW = B // NW


def _sc_sort_body(noise_hbm, out_remain, out_masked, out_revert,
                  ka, va, kb, vb, hist, offs, revert_v):
    wid = lax.axis_index("s") * NC + lax.axis_index("c")

    def row_body(r, carry):
        b = wid * RPW + r
        pltpu.sync_copy(noise_hbm.at[pl.ds(b * SEQ, SEQ)], ka)

        def init_chunk(c, carry):
            va[pl.ds(c * LN, LN)] = lax.iota(jnp.int32, LN) + c * LN
            return carry
        lax.fori_loop(0, CH, init_chunk, 0)

        for p, (ks, vs, kd, vd) in enumerate(
            ((ka, va, kb, vb), (kb, vb, ka, va), (ka, va, kb, vb))):
            shift = 10 * p

            def clear_chunk(c, carry):
                hist[pl.ds(c * LN, LN)] = jnp.zeros((LN,), jnp.int32)
                return carry
            lax.fori_loop(0, NB // LN, clear_chunk, 0)

            def hist_chunk(c, carry, ks=ks, shift=shift):
                k16 = ks[pl.ds(c * LN, LN)]
                d = (k16 >> shift) & (NB - 1)
                cnt, last = plsc.scan_count(d)
                plsc.addupdate_scatter(hist, [d], cnt, mask=last)
                return carry
            lax.fori_loop(0, CH, hist_chunk, 0)

            def scan_chunk(c, run):
                h = hist[pl.ds(c * LN, LN)]
                cs = plsc.cumsum(h)
                offs[pl.ds(c * LN, LN)] = cs - h + run
                return run + jnp.sum(h)
            lax.fori_loop(0, NB // LN, scan_chunk, jnp.int32(0))

            def perm_chunk(c, carry, ks=ks, vs=vs, kd=kd, vd=vd, shift=shift):
                k16 = ks[pl.ds(c * LN, LN)]
                v16 = vs[pl.ds(c * LN, LN)]
                d = (k16 >> shift) & (NB - 1)
                cnt, last = plsc.scan_count(d)
                base = plsc.load_gather(offs, [d])
                pos = base + cnt - 1
                plsc.store_scatter(kd, [pos], k16)
                plsc.store_scatter(vd, [pos], v16)
                plsc.addupdate_scatter(offs, [d], cnt, mask=last)
                return carry
            lax.fori_loop(0, CH, perm_chunk, 0)

        def rev_chunk(c, carry):
            v16 = vb[pl.ds(c * LN, LN)]
            plsc.store_scatter(revert_v, [v16], lax.iota(jnp.int32, LN) + c * LN)
            return carry
        lax.fori_loop(0, CH, rev_chunk, 0)

        pltpu.sync_copy(revert_v, out_revert.at[pl.ds(b * SEQ, SEQ)])
        pltpu.sync_copy(vb.at[pl.ds(0, NR)], out_remain.at[pl.ds(b * NR, NR)])
        pltpu.sync_copy(vb.at[pl.ds(NR, NM)], out_masked.at[pl.ds(b * NM, NM)])
        return carry

    lax.fori_loop(0, RPW, row_body, 0)


DP = 256


def _tc_transpose_body(in_ref, out_ref):
    x = in_ref[...]
    y = jnp.transpose(x, (0, 2, 1))
    pad = jnp.zeros(y.shape[:2] + (DP - D,), jnp.float32)
    out_ref[...] = jnp.concatenate([y, pad], axis=2)


KP = 264


def _sc_gather_body(data_hbm, remain_hbm, out_data, idx_v, mi, mi2, rows_v,
                    sem):
    wid = lax.axis_index("s") * NC + lax.axis_index("c")

    def row_body(r, carry):
        b = wid * RPW + r
        pltpu.sync_copy(remain_hbm.at[pl.ds(b * NR, NR)], idx_v)

        iota = lax.iota(jnp.int32, LN)
        bvec = jnp.zeros((LN,), jnp.int32) + b
        mi2[...] = bvec
        zeros = jnp.zeros((LN,), jnp.int32)
        plsc.store_scatter(mi, [zeros, zeros], bvec, mask=iota == 0)

        def mi_chunk(c, carry):
            idx16 = idx_v[pl.ds(c * LN, LN)]
            g16 = ((idx16 + 1) << 7) + b
            pos = iota + (c * LN + 1)
            r16 = pos >> 7
            c16 = pos & 127
            plsc.store_scatter(mi, [r16, c16], g16, mask=pos <= 255)
            plsc.store_scatter(mi2, [jnp.maximum(pos - NR, 0)], g16,
                              mask=pos == NR)
            return carry
        lax.fori_loop(0, NR // LN, mi_chunk, 0)

        cp0 = pltpu.async_copy(data_hbm.at[mi.at[0]],
                               rows_v.at[pl.ds(0, 128)], sem)
        cp1 = pltpu.async_copy(data_hbm.at[mi.at[1]],
                               rows_v.at[pl.ds(128, 128)], sem)
        cp2 = pltpu.async_copy(data_hbm.at[mi2.at[pl.ds(0, 8)]],
                               rows_v.at[pl.ds(256, 8)], sem)
        cp0.wait()
        cp1.wait()
        cp2.wait()
        pltpu.sync_copy(rows_v, out_data.at[b])
        return carry

    lax.fori_loop(0, RPW, row_body, 0)


def kernel(data, noise):
    noise_f = lax.bitcast_convert_type(noise, jnp.int32).reshape(B * SEQ)

    mesh = plsc.VectorSubcoreMesh(core_axis_name="c", subcore_axis_name="s")
    out_type = [
        jax.ShapeDtypeStruct((B * NR,), jnp.int32),
        jax.ShapeDtypeStruct((B * NM,), jnp.int32),
        jax.ShapeDtypeStruct((B * SEQ,), jnp.int32),
    ]
    scratch = [
        pltpu.VMEM((SEQ,), jnp.int32),
        pltpu.VMEM((SEQ,), jnp.int32),
        pltpu.VMEM((SEQ,), jnp.int32),
        pltpu.VMEM((SEQ,), jnp.int32),
        pltpu.VMEM((NB,), jnp.int32),
        pltpu.VMEM((NB,), jnp.int32),
        pltpu.VMEM((SEQ,), jnp.int32),
    ]
    sort_run = pl.kernel(_sc_sort_body, out_type=out_type, mesh=mesh,
                         scratch_types=scratch,
                         compiler_params=pltpu.CompilerParams(
                             needs_layout_passes=False,
                             use_tc_tiling_on_sc=False))
    orem, omask, orev = sort_run(noise_f)

    remain_idx = orem.reshape(B, NR)
    masked_idx = omask.reshape(B, NM)
    revert_idx = orev.reshape(B, SEQ)

    data_t = jnp.transpose(data, (1, 2, 0))
    J = 41
    data_rows = pl.pallas_call(
        _tc_transpose_body,
        grid=(L // J,),
        in_specs=[pl.BlockSpec((J, D, B), lambda j: (j, 0, 0))],
        out_specs=pl.BlockSpec((J, B, DP), lambda j: (j, 0, 0)),
        out_shape=jax.ShapeDtypeStruct((L, B, DP), jnp.float32),
    )(data_t).reshape(L * B, DP)

    gather_run = pl.kernel(
        _sc_gather_body,
        out_type=jax.ShapeDtypeStruct((B, KP, DP), jnp.float32),
        mesh=mesh,
        scratch_types=[
            pltpu.VMEM((NR,), jnp.int32),
            pltpu.VMEM((2, 128), jnp.int32),
            pltpu.VMEM((LN,), jnp.int32),
            pltpu.VMEM((KP, DP), jnp.float32),
            pltpu.SemaphoreType.DMA,
        ],
        compiler_params=pltpu.CompilerParams(
            needs_layout_passes=False,
            use_tc_tiling_on_sc=True))
    out_pad = gather_run(data_rows, orem)

    total_remain_data = out_pad[:, :NR + 1, :D]

    total_remain_padding_mask = jnp.ones((B, NR + 1), jnp.float32)
    revert_padding_mask = jnp.ones((B, L), jnp.float32)
    return (total_remain_data, remain_idx, masked_idx, revert_idx,
            total_remain_padding_mask, revert_padding_mask)
